# baseline (device time: 233753 ns/iter reference)
import jax
import jax.numpy as jnp
from jax import lax
from jax.experimental import pallas as pl
from jax.experimental.pallas import tpu as pltpu

N_DEV = 16
NSLOTS = 4
SUB = 4

RING = [0, 1, 5, 4, 8, 9, 13, 12, 15, 14, 10, 11, 7, 6, 2, 3]
INV = [0] * N_DEV
for _r, _lg in enumerate(RING):
    INV[_lg] = _r


def kernel(x, w_mat, scale_x, scale_w):
    m, _ = x.shape
    _, n = w_mat.shape
    ch = m // N_DEV
    hn = n // 2
    sb = ch // SUB

    my_log = lax.axis_index("i")
    ring_arr = jnp.asarray(RING, dtype=jnp.int32)
    inv_arr = jnp.asarray(INV, dtype=jnp.int32)
    kpos = inv_arr[my_log]
    right_log = ring_arr[(kpos + 1) % N_DEV]
    left_log = ring_arr[(kpos + N_DEV - 1) % N_DEV]
    pos = jnp.stack([kpos, left_log, right_log]).astype(jnp.int32)
    pos = pos.reshape(3, 1)

    def body(x_ref, w_ref, sx_ref, sw_ref, pos_ref, out_ref, part_ref,
             rs_comm_cw, rs_comm_ccw, ag_comm_cw, ag_comm_ccw,
             rs_send_cw, rs_recv_cw, rs_send_ccw, rs_recv_ccw,
             ag_send_cw, ag_recv_cw, ag_send_ccw, ag_recv_ccw,
             credits):
        my = pos_ref[0, 0]
        left = pos_ref[1, 0]
        right = pos_ref[2, 0]

        barrier = pltpu.get_barrier_semaphore()
        for nbr in (left, right):
            pl.semaphore_signal(barrier, inc=1, device_id=(nbr,),
                                device_id_type=pl.DeviceIdType.MESH)
        pl.semaphore_wait(barrier, 2)

        part_ref[...] = jnp.dot(
            x_ref[...], w_ref[...], preferred_element_type=jnp.float32
        ).astype(jnp.bfloat16)

        def subrows(c, b):
            return pl.ds(c * ch + b * sb, sb)

        flows = []
        for b in range(SUB):
            flows.append(dict(
                b=b, half=pl.ds(0, hn), to=right, sgn=+1,
                comm=rs_comm_cw, send=rs_send_cw, recv=rs_recv_cw,
                agcomm=ag_comm_cw, agsend=ag_send_cw, agrecv=ag_recv_cw,
                rs_credit=0 * SUB + b, ag_credit=2 * SUB + b,
                credit_to=left,
            ))
            flows.append(dict(
                b=b, half=pl.ds(hn, hn), to=left, sgn=-1,
                comm=rs_comm_ccw, send=rs_send_ccw, recv=rs_recv_ccw,
                agcomm=ag_comm_ccw, agsend=ag_send_ccw, agrecv=ag_recv_ccw,
                rs_credit=1 * SUB + b, ag_credit=3 * SUB + b,
                credit_to=right,
            ))

        def rs_rdma(f, s):
            c = (my + f["sgn"] * (-s) + N_DEV) % N_DEV
            return pltpu.make_async_remote_copy(
                src_ref=part_ref.at[subrows(c, f["b"]), f["half"]],
                dst_ref=f["comm"].at[f["b"], s % NSLOTS],
                send_sem=f["send"].at[f["b"], s],
                recv_sem=f["recv"].at[f["b"], s],
                device_id=(f["to"],),
                device_id_type=pl.DeviceIdType.MESH,
            )

        for f in flows:
            rs_rdma(f, 0).start()
        for s in range(N_DEV - 1):
            for f in flows:
                rs_rdma(f, s).wait()
                ac = (my + f["sgn"] * (-s - 1) + N_DEV) % N_DEV
                part_ref[subrows(ac, f["b"]), f["half"]] = (
                    part_ref[subrows(ac, f["b"]), f["half"]]
                    + f["comm"][f["b"], s % NSLOTS]
                )
                if s < N_DEV - 2:
                    if s + 1 >= NSLOTS:
                        pl.semaphore_wait(credits.at[f["rs_credit"]], 1)
                    rs_rdma(f, s + 1).start()
                if s < (N_DEV - 1) - NSLOTS:
                    pl.semaphore_signal(
                        credits.at[f["rs_credit"]], inc=1,
                        device_id=(f["credit_to"],),
                        device_id_type=pl.DeviceIdType.MESH)

        scale = sx_ref[0, 0] * sw_ref[0, 0]
        own_cw = (my + 1) % N_DEV
        own_ccw = (my - 1 + N_DEV) % N_DEV
        for own, half in ((own_cw, pl.ds(0, hn)), (own_ccw, pl.ds(hn, hn))):
            rws = pl.ds(own * ch, ch)
            v = jnp.maximum(
                part_ref[rws, half].astype(jnp.float32) * scale, 0.0)
            out_ref[rws, half] = v
            part_ref[rws, half] = v.astype(jnp.bfloat16)

        def ag_rdma(f, s):
            own = (my + f["sgn"] + N_DEV) % N_DEV
            if s == 0:
                src = part_ref.at[subrows(own, f["b"]), f["half"]]
            else:
                src = f["agcomm"].at[f["b"], (s - 1) % NSLOTS]
            return pltpu.make_async_remote_copy(
                src_ref=src,
                dst_ref=f["agcomm"].at[f["b"], s % NSLOTS],
                send_sem=f["agsend"].at[f["b"], s],
                recv_sem=f["agrecv"].at[f["b"], s],
                device_id=(f["to"],),
                device_id_type=pl.DeviceIdType.MESH,
            )

        for f in flows:
            ag_rdma(f, 0).start()
        for s in range(N_DEV - 1):
            for f in flows:
                ag_rdma(f, s).wait()
                if s < N_DEV - 2:
                    if s + 1 >= NSLOTS:
                        pl.semaphore_wait(credits.at[f["ag_credit"]], 1)
                    ag_rdma(f, s + 1).start()
                gc = (my + f["sgn"] * (-s) + N_DEV) % N_DEV
                out_ref[subrows(gc, f["b"]), f["half"]] = (
                    f["agcomm"][f["b"], s % NSLOTS].astype(jnp.float32))
                if 1 <= s <= (N_DEV - 1) - NSLOTS:
                    pl.semaphore_signal(
                        credits.at[f["ag_credit"]], inc=1,
                        device_id=(f["credit_to"],),
                        device_id_type=pl.DeviceIdType.MESH)

    dma2 = pltpu.SemaphoreType.DMA((SUB, N_DEV - 1))
    return pl.pallas_call(
        body,
        out_shape=jax.ShapeDtypeStruct((m, n), jnp.float32),
        in_specs=[
            pl.BlockSpec(memory_space=pltpu.VMEM),
            pl.BlockSpec(memory_space=pltpu.VMEM),
            pl.BlockSpec(memory_space=pltpu.SMEM),
            pl.BlockSpec(memory_space=pltpu.SMEM),
            pl.BlockSpec(memory_space=pltpu.SMEM),
        ],
        out_specs=pl.BlockSpec(memory_space=pltpu.VMEM),
        scratch_shapes=[
            pltpu.VMEM((m, n), jnp.bfloat16),
            pltpu.VMEM((SUB, NSLOTS, sb, hn), jnp.bfloat16),
            pltpu.VMEM((SUB, NSLOTS, sb, hn), jnp.bfloat16),
            pltpu.VMEM((SUB, NSLOTS, sb, hn), jnp.bfloat16),
            pltpu.VMEM((SUB, NSLOTS, sb, hn), jnp.bfloat16),
            dma2, dma2, dma2, dma2,
            dma2, dma2, dma2, dma2,
            pltpu.SemaphoreType.REGULAR((4 * SUB,)),
        ],
        compiler_params=pltpu.CompilerParams(
            collective_id=0,
            vmem_limit_bytes=120 * 1024 * 1024,
        ),
    )(x.astype(jnp.bfloat16), w_mat.astype(jnp.bfloat16),
      scale_x.reshape(1, 1), scale_w.reshape(1, 1), pos)


# device time: 227008 ns/iter; 1.0297x vs baseline; 1.0297x over previous
import jax
import jax.numpy as jnp
from jax import lax
from jax.experimental import pallas as pl
from jax.experimental.pallas import tpu as pltpu

N_DEV = 16
NSLOTS = 4
SUB = 2

RING = [0, 1, 5, 4, 8, 9, 13, 12, 15, 14, 10, 11, 7, 6, 2, 3]
INV = [0] * N_DEV
for _r, _lg in enumerate(RING):
    INV[_lg] = _r


def kernel(x, w_mat, scale_x, scale_w):
    m, _ = x.shape
    _, n = w_mat.shape
    ch = m // N_DEV
    hn = n // 2
    sb = ch // SUB

    my_log = lax.axis_index("i")
    ring_arr = jnp.asarray(RING, dtype=jnp.int32)
    inv_arr = jnp.asarray(INV, dtype=jnp.int32)
    kpos = inv_arr[my_log]
    right_log = ring_arr[(kpos + 1) % N_DEV]
    left_log = ring_arr[(kpos + N_DEV - 1) % N_DEV]
    pos = jnp.stack([kpos, left_log, right_log]).astype(jnp.int32)
    pos = pos.reshape(3, 1)

    def body(x_ref, w_ref, sx_ref, sw_ref, pos_ref, out_ref, part_ref,
             rs_comm_cw, rs_comm_ccw, ag_comm_cw, ag_comm_ccw,
             rs_send_cw, rs_recv_cw, rs_send_ccw, rs_recv_ccw,
             ag_send_cw, ag_recv_cw, ag_send_ccw, ag_recv_ccw,
             credits):
        my = pos_ref[0, 0]
        left = pos_ref[1, 0]
        right = pos_ref[2, 0]

        barrier = pltpu.get_barrier_semaphore()
        for nbr in (left, right):
            pl.semaphore_signal(barrier, inc=1, device_id=(nbr,),
                                device_id_type=pl.DeviceIdType.MESH)
        pl.semaphore_wait(barrier, 2)

        def compute_chunk(c):
            part_ref[pl.ds(c * ch, ch), :] = jnp.dot(
                x_ref[pl.ds(c * ch, ch), :], w_ref[...],
                preferred_element_type=jnp.float32,
            ).astype(jnp.bfloat16)

        compute_chunk(my)

        def subrows(c, b):
            return pl.ds(c * ch + b * sb, sb)

        flows = []
        for b in range(SUB):
            flows.append(dict(
                b=b, half=pl.ds(0, hn), to=right, sgn=+1,
                comm=rs_comm_cw, send=rs_send_cw, recv=rs_recv_cw,
                agcomm=ag_comm_cw, agsend=ag_send_cw, agrecv=ag_recv_cw,
                rs_credit=0 * SUB + b, ag_credit=2 * SUB + b,
                credit_to=left,
            ))
            flows.append(dict(
                b=b, half=pl.ds(hn, hn), to=left, sgn=-1,
                comm=rs_comm_ccw, send=rs_send_ccw, recv=rs_recv_ccw,
                agcomm=ag_comm_ccw, agsend=ag_send_ccw, agrecv=ag_recv_ccw,
                rs_credit=1 * SUB + b, ag_credit=3 * SUB + b,
                credit_to=right,
            ))

        def rs_rdma(f, s):
            c = (my + f["sgn"] * (-s) + N_DEV) % N_DEV
            return pltpu.make_async_remote_copy(
                src_ref=part_ref.at[subrows(c, f["b"]), f["half"]],
                dst_ref=f["comm"].at[f["b"], s % NSLOTS],
                send_sem=f["send"].at[f["b"], s],
                recv_sem=f["recv"].at[f["b"], s],
                device_id=(f["to"],),
                device_id_type=pl.DeviceIdType.MESH,
            )

        for f in flows:
            rs_rdma(f, 0).start()
        for s in range(N_DEV - 1):
            if s + 1 <= N_DEV // 2 - 1:
                compute_chunk((my - s - 1 + N_DEV) % N_DEV)
                compute_chunk((my + s + 1) % N_DEV)
            elif s + 1 == N_DEV // 2:
                compute_chunk((my + N_DEV // 2) % N_DEV)
            for f in flows:
                rs_rdma(f, s).wait()
                ac = (my + f["sgn"] * (-s - 1) + N_DEV) % N_DEV
                part_ref[subrows(ac, f["b"]), f["half"]] = (
                    part_ref[subrows(ac, f["b"]), f["half"]]
                    + f["comm"][f["b"], s % NSLOTS]
                )
                if s < N_DEV - 2:
                    if s + 1 >= NSLOTS:
                        pl.semaphore_wait(credits.at[f["rs_credit"]], 1)
                    rs_rdma(f, s + 1).start()
                if s < (N_DEV - 1) - NSLOTS:
                    pl.semaphore_signal(
                        credits.at[f["rs_credit"]], inc=1,
                        device_id=(f["credit_to"],),
                        device_id_type=pl.DeviceIdType.MESH)

        scale = sx_ref[0, 0] * sw_ref[0, 0]
        own_cw = (my + 1) % N_DEV
        own_ccw = (my - 1 + N_DEV) % N_DEV
        for own, half in ((own_cw, pl.ds(0, hn)), (own_ccw, pl.ds(hn, hn))):
            rws = pl.ds(own * ch, ch)
            v = jnp.maximum(
                part_ref[rws, half].astype(jnp.float32) * scale, 0.0)
            out_ref[rws, half] = v
            part_ref[rws, half] = v.astype(jnp.bfloat16)

        def ag_rdma(f, s):
            own = (my + f["sgn"] + N_DEV) % N_DEV
            if s == 0:
                src = part_ref.at[subrows(own, f["b"]), f["half"]]
            else:
                src = f["agcomm"].at[f["b"], (s - 1) % NSLOTS]
            return pltpu.make_async_remote_copy(
                src_ref=src,
                dst_ref=f["agcomm"].at[f["b"], s % NSLOTS],
                send_sem=f["agsend"].at[f["b"], s],
                recv_sem=f["agrecv"].at[f["b"], s],
                device_id=(f["to"],),
                device_id_type=pl.DeviceIdType.MESH,
            )

        for f in flows:
            ag_rdma(f, 0).start()
        for s in range(N_DEV - 1):
            for f in flows:
                ag_rdma(f, s).wait()
                if s < N_DEV - 2:
                    if s + 1 >= NSLOTS:
                        pl.semaphore_wait(credits.at[f["ag_credit"]], 1)
                    ag_rdma(f, s + 1).start()
                gc = (my + f["sgn"] * (-s) + N_DEV) % N_DEV
                out_ref[subrows(gc, f["b"]), f["half"]] = (
                    f["agcomm"][f["b"], s % NSLOTS].astype(jnp.float32))
                if 1 <= s <= (N_DEV - 1) - NSLOTS:
                    pl.semaphore_signal(
                        credits.at[f["ag_credit"]], inc=1,
                        device_id=(f["credit_to"],),
                        device_id_type=pl.DeviceIdType.MESH)

    dma2 = pltpu.SemaphoreType.DMA((SUB, N_DEV - 1))
    return pl.pallas_call(
        body,
        out_shape=jax.ShapeDtypeStruct((m, n), jnp.float32),
        in_specs=[
            pl.BlockSpec(memory_space=pltpu.VMEM),
            pl.BlockSpec(memory_space=pltpu.VMEM),
            pl.BlockSpec(memory_space=pltpu.SMEM),
            pl.BlockSpec(memory_space=pltpu.SMEM),
            pl.BlockSpec(memory_space=pltpu.SMEM),
        ],
        out_specs=pl.BlockSpec(memory_space=pltpu.VMEM),
        scratch_shapes=[
            pltpu.VMEM((m, n), jnp.bfloat16),
            pltpu.VMEM((SUB, NSLOTS, sb, hn), jnp.bfloat16),
            pltpu.VMEM((SUB, NSLOTS, sb, hn), jnp.bfloat16),
            pltpu.VMEM((SUB, NSLOTS, sb, hn), jnp.bfloat16),
            pltpu.VMEM((SUB, NSLOTS, sb, hn), jnp.bfloat16),
            dma2, dma2, dma2, dma2,
            dma2, dma2, dma2, dma2,
            pltpu.SemaphoreType.REGULAR((4 * SUB,)),
        ],
        compiler_params=pltpu.CompilerParams(
            collective_id=0,
            vmem_limit_bytes=120 * 1024 * 1024,
        ),
    )(x.astype(jnp.bfloat16), w_mat.astype(jnp.bfloat16),
      scale_x.reshape(1, 1), scale_w.reshape(1, 1), pos)
